# baseline (speedup 1.0000x reference)
import functools

import jax
import jax.numpy as jnp
from jax import lax
from jax.experimental import pallas as pl
from jax.experimental.pallas import tpu as pltpu
from jax.experimental.pallas import tpu_sc as plsc

N = 16384
C = 1000
L = 16
NC = 2
NS = 16
NW = NC * NS
ROWS_PER_W = N // NW
CHUNK = 128
NCHUNK = ROWS_PER_W // CHUNK

_mesh = plsc.VectorSubcoreMesh(core_axis_name="c", subcore_axis_name="s")


@functools.partial(
    pl.kernel,
    mesh=_mesh,
    out_type=[
        jax.ShapeDtypeStruct((NC, L), jnp.float32),
        jax.ShapeDtypeStruct((NC, NS * L), jnp.float32),
    ],
    scratch_types=[
        pltpu.VMEM((ROWS_PER_W,), jnp.int32),
        pltpu.VMEM((ROWS_PER_W,), jnp.float32),
        pltpu.VMEM((CHUNK,), jnp.int32),
        pltpu.VMEM((CHUNK,), jnp.int32),
        pltpu.VMEM((CHUNK,), jnp.int32),
        pltpu.VMEM((CHUNK,), jnp.int32),
        pltpu.VMEM((CHUNK,), jnp.float32),
        pltpu.VMEM((CHUNK,), jnp.float32),
        pltpu.VMEM((CHUNK,), jnp.float32),
        pltpu.VMEM((CHUNK,), jnp.float32),
        pltpu.VMEM((L,), jnp.float32),
        pltpu.VMEM((NS * L,), jnp.float32),
        pltpu.SemaphoreType.DMA,
    ],
)
def _gan_loss_sc(prob_flat_hbm, tgt_hbm, rwd_hbm, out_hbm, stage_hbm,
                 tgt_v, rwd_v, idx0, idx1, idx2, idx3,
                 sel0, sel1, sel2, sel3,
                 part_v, allp_v, sem):
    idx_bufs = (idx0, idx1, idx2, idx3)
    sel_bufs = (sel0, sel1, sel2, sel3)

    sid = lax.axis_index("s")
    cid = lax.axis_index("c")
    wid = sid * NC + cid
    base = wid * ROWS_PER_W

    pltpu.sync_copy(tgt_hbm.at[pl.ds(base, ROWS_PER_W)], tgt_v)
    pltpu.sync_copy(rwd_hbm.at[pl.ds(base, ROWS_PER_W)], rwd_v)

    lane = lax.broadcasted_iota(jnp.int32, (L,), 0)
    for k in range(NCHUNK):
        for j in range(CHUNK // L):
            off = k * CHUNK + j * L
            row = (base + off) + lane
            t = tgt_v[pl.ds(off, L)]
            idx_bufs[k][pl.ds(j * L, L)] = row * C + t

    copies = [
        pltpu.async_copy(prob_flat_hbm.at[idx_bufs[k]], sel_bufs[k], sem)
        for k in range(NCHUNK)
    ]
    for cp in copies:
        cp.wait()

    acc = jnp.zeros((L,), jnp.float32)
    for k in range(NCHUNK):
        for j in range(CHUNK // L):
            off = k * CHUNK + j * L
            s = sel_bufs[k][pl.ds(j * L, L)]
            r = rwd_v[pl.ds(off, L)]
            acc = acc + s * (1.0 - r + 1e-6)
    part_v[...] = acc

    pltpu.sync_copy(part_v, stage_hbm.at[cid, pl.ds(sid * L, L)])
    plsc.subcore_barrier()

    @pl.when(sid == 0)
    def _():
        pltpu.sync_copy(stage_hbm.at[cid], allp_v)
        tot = jnp.zeros((L,), jnp.float32)
        for w in range(NS):
            tot = tot + allp_v[pl.ds(w * L, L)]
        scalar = tot[0]
        for i in range(1, L):
            scalar = scalar + tot[i]
        scalar = scalar * (-1.0 / N)
        out_v = part_v
        out_v[...] = jnp.zeros((L,), jnp.float32) + scalar
        pltpu.sync_copy(out_v, out_hbm.at[cid])


def kernel(prob, target, reward):
    prob_flat = prob.reshape(-1)
    tgt = target.astype(jnp.int32)
    rwd = reward.astype(jnp.float32)
    out, _ = _gan_loss_sc(prob_flat, tgt, rwd)
    return out[0, 0] + out[1, 0]


# speedup vs baseline: 5.8882x; 5.8882x over previous
import functools

import jax
import jax.numpy as jnp
from jax import lax
from jax.experimental import pallas as pl
from jax.experimental.pallas import tpu as pltpu
from jax.experimental.pallas import tpu_sc as plsc

N = 16384
C = 1000
L = 16
NC = 2
NS = 16
NW = NC * NS
ROWS_PER_W = N // NW
CHUNK = 128
NCHUNK = ROWS_PER_W // CHUNK

_mesh = plsc.VectorSubcoreMesh(core_axis_name="c", subcore_axis_name="s")


@functools.partial(
    pl.kernel,
    mesh=_mesh,
    out_type=[
        jax.ShapeDtypeStruct((NC, L), jnp.float32),
        jax.ShapeDtypeStruct((NC, NS * L), jnp.float32),
    ],
    scratch_types=[
        pltpu.VMEM((ROWS_PER_W,), jnp.int32),
        pltpu.VMEM((ROWS_PER_W,), jnp.float32),
        pltpu.VMEM((CHUNK,), jnp.int32),
        pltpu.VMEM((CHUNK,), jnp.int32),
        pltpu.VMEM((CHUNK,), jnp.int32),
        pltpu.VMEM((CHUNK,), jnp.int32),
        pltpu.VMEM((CHUNK,), jnp.float32),
        pltpu.VMEM((CHUNK,), jnp.float32),
        pltpu.VMEM((CHUNK,), jnp.float32),
        pltpu.VMEM((CHUNK,), jnp.float32),
        pltpu.VMEM((L,), jnp.float32),
        pltpu.VMEM((NS * L,), jnp.float32),
        pltpu.SemaphoreType.DMA,
    ],
)
def _gan_loss_sc(prob_hbm, tgt_hbm, rwd_hbm, out_hbm, stage_hbm,
                 tgt_v, rwd_v, idx0, idx1, idx2, idx3,
                 sel0, sel1, sel2, sel3,
                 part_v, allp_v, sem):
    idx_bufs = (idx0, idx1, idx2, idx3)
    sel_bufs = (sel0, sel1, sel2, sel3)
    prob_flat_hbm = prob_hbm

    sid = lax.axis_index("s")
    cid = lax.axis_index("c")
    wid = sid * NC + cid
    base = wid * ROWS_PER_W

    pltpu.sync_copy(tgt_hbm.at[pl.ds(base, ROWS_PER_W)], tgt_v)
    pltpu.sync_copy(rwd_hbm.at[pl.ds(base, ROWS_PER_W)], rwd_v)

    lane = lax.broadcasted_iota(jnp.int32, (L,), 0)
    for k in range(NCHUNK):
        for j in range(CHUNK // L):
            off = k * CHUNK + j * L
            row = (base + off) + lane
            t = tgt_v[pl.ds(off, L)]
            idx_bufs[k][pl.ds(j * L, L)] = (
                (t >> 3) * 131072 + (row >> 7) * 1024 + (t & 7) * 128 + (row & 127)
            )

    copies = [
        pltpu.async_copy(prob_flat_hbm.at[idx_bufs[k]], sel_bufs[k], sem)
        for k in range(NCHUNK)
    ]
    for cp in copies:
        cp.wait()

    acc = jnp.zeros((L,), jnp.float32)
    for k in range(NCHUNK):
        for j in range(CHUNK // L):
            off = k * CHUNK + j * L
            s = sel_bufs[k][pl.ds(j * L, L)]
            r = rwd_v[pl.ds(off, L)]
            acc = acc + s * (1.0 - r + 1e-6)
    part_v[...] = acc

    pltpu.sync_copy(part_v, stage_hbm.at[cid, pl.ds(sid * L, L)])
    plsc.subcore_barrier()

    @pl.when(sid == 0)
    def _():
        pltpu.sync_copy(stage_hbm.at[cid], allp_v)
        tot = jnp.zeros((L,), jnp.float32)
        for w in range(NS):
            tot = tot + allp_v[pl.ds(w * L, L)]
        scalar = tot[0]
        for i in range(1, L):
            scalar = scalar + tot[i]
        scalar = scalar * (-1.0 / N)
        out_v = part_v
        out_v[...] = jnp.zeros((L,), jnp.float32) + scalar
        pltpu.sync_copy(out_v, out_hbm.at[cid])


def kernel(prob, target, reward):
    tgt = target.astype(jnp.int32)
    rwd = reward.astype(jnp.float32)
    prob_phys = prob.reshape(128, 128, 125, 8).transpose(2, 0, 3, 1).reshape(-1)
    out, _ = _gan_loss_sc(prob_phys, tgt, rwd)
    return out[0, 0] + out[1, 0]


# speedup vs baseline: 6.2951x; 1.0691x over previous
import functools

import jax
import jax.numpy as jnp
from jax import lax
from jax.experimental import pallas as pl
from jax.experimental.pallas import tpu as pltpu
from jax.experimental.pallas import tpu_sc as plsc

N = 16384
C = 1000
L = 16
NC = 2
NS = 16
NW = NC * NS
ROWS_PER_W = N // NW
CHUNK = 128
NCHUNK = ROWS_PER_W // CHUNK

_mesh = plsc.VectorSubcoreMesh(core_axis_name="c", subcore_axis_name="s")


@functools.partial(
    pl.kernel,
    mesh=_mesh,
    out_type=jax.ShapeDtypeStruct((NW, L), jnp.float32),
    scratch_types=[
        pltpu.VMEM((ROWS_PER_W,), jnp.int32),
        pltpu.VMEM((ROWS_PER_W,), jnp.float32),
        pltpu.VMEM((ROWS_PER_W,), jnp.int32),
        pltpu.VMEM((ROWS_PER_W,), jnp.float32),
        pltpu.VMEM((L,), jnp.float32),
        pltpu.SemaphoreType.DMA,
        pltpu.SemaphoreType.DMA,
    ],
)
def _gan_loss_sc(prob_flat_hbm, tgt_hbm, rwd_hbm, out_hbm,
                 tgt_v, rwd_v, idx_v, sel_v, part_v, sem_in, sem_g):
    sid = lax.axis_index("s")
    cid = lax.axis_index("c")
    wid = sid * NC + cid
    base = wid * ROWS_PER_W

    cp_t = pltpu.async_copy(tgt_hbm.at[pl.ds(base, ROWS_PER_W)], tgt_v, sem_in)
    cp_r = pltpu.async_copy(rwd_hbm.at[pl.ds(base, ROWS_PER_W)], rwd_v, sem_in)
    cp_t.wait()

    lane = lax.broadcasted_iota(jnp.int32, (L,), 0)
    gathers = []
    for k in range(NCHUNK):
        for j in range(CHUNK // L):
            off = k * CHUNK + j * L
            s = ((base + off) >> 7) * 1024 + ((base + off) & 127)
            t = tgt_v[pl.ds(off, L)]
            idx_v[pl.ds(off, L)] = (
                (t >> 3) * 131072 + (t & 7) * 128 + s + lane
            )
        gathers.append(
            pltpu.async_copy(
                prob_flat_hbm.at[idx_v.at[pl.ds(k * CHUNK, CHUNK)]],
                sel_v.at[pl.ds(k * CHUNK, CHUNK)],
                sem_g,
            )
        )

    cp_r.wait()
    for g in gathers:
        g.wait()

    acc = jnp.zeros((L,), jnp.float32)
    one = jnp.full((L,), 1.0 + 1e-6, jnp.float32)
    for k in range(NCHUNK):
        for j in range(CHUNK // L):
            off = k * CHUNK + j * L
            acc = acc + sel_v[pl.ds(off, L)] * (one - rwd_v[pl.ds(off, L)])
    part_v[...] = acc
    pltpu.sync_copy(part_v, out_hbm.at[wid])


def kernel(prob, target, reward):
    tgt = target.astype(jnp.int32)
    rwd = reward.astype(jnp.float32)
    prob_phys = prob.reshape(128, 128, 125, 8).transpose(2, 0, 3, 1).reshape(-1)
    partials = _gan_loss_sc(prob_phys, tgt, rwd)
    return jnp.sum(partials) * (-1.0 / N)


# speedup vs baseline: 6.3139x; 1.0030x over previous
import functools

import jax
import jax.numpy as jnp
from jax import lax
from jax.experimental import pallas as pl
from jax.experimental.pallas import tpu as pltpu
from jax.experimental.pallas import tpu_sc as plsc

N = 16384
C = 1000
L = 16
NC = 2
NS = 16
NW = NC * NS
ROWS_PER_W = N // NW
CHUNK = 128
NCHUNK = ROWS_PER_W // CHUNK

_mesh = plsc.VectorSubcoreMesh(core_axis_name="c", subcore_axis_name="s")


@functools.partial(
    pl.kernel,
    mesh=_mesh,
    out_type=jax.ShapeDtypeStruct((NW, L), jnp.float32),
    scratch_types=[
        pltpu.VMEM((ROWS_PER_W,), jnp.int32),
        pltpu.VMEM((ROWS_PER_W,), jnp.float32),
        pltpu.VMEM((ROWS_PER_W,), jnp.int32),
        pltpu.VMEM((ROWS_PER_W,), jnp.float32),
        pltpu.VMEM((L,), jnp.float32),
        pltpu.SemaphoreType.DMA,
        pltpu.SemaphoreType.DMA,
    ],
)
def _gan_loss_sc(prob_flat_hbm, tgt_hbm, rwd_hbm, out_hbm,
                 tgt_v, rwd_v, idx_v, sel_v, part_v, sem_in, sem_g):
    sid = lax.axis_index("s")
    cid = lax.axis_index("c")
    wid = sid * NC + cid
    base = wid * ROWS_PER_W

    cp_t = pltpu.async_copy(tgt_hbm.at[pl.ds(base, ROWS_PER_W)], tgt_v, sem_in)
    cp_r = pltpu.async_copy(rwd_hbm.at[pl.ds(base, ROWS_PER_W)], rwd_v, sem_in)
    cp_t.wait()

    lane = lax.broadcasted_iota(jnp.int32, (L,), 0)
    gathers = []
    for k in range(NCHUNK):
        def idx_body(j, _, k=k):
            off = k * CHUNK + j * L
            s = ((base + off) >> 7) * 1024 + ((base + off) & 127)
            t = tgt_v[pl.ds(off, L)]
            idx_v[pl.ds(off, L)] = (
                (t >> 3) * 131072 + (t & 7) * 128 + s + lane
            )
            return 0
        lax.fori_loop(0, CHUNK // L, idx_body, 0, unroll=2)
        gathers.append(
            pltpu.async_copy(
                prob_flat_hbm.at[idx_v.at[pl.ds(k * CHUNK, CHUNK)]],
                sel_v.at[pl.ds(k * CHUNK, CHUNK)],
                sem_g,
            )
        )

    cp_r.wait()
    for g in gathers:
        g.wait()

    one = jnp.full((L,), 1.0 + 1e-6, jnp.float32)

    def acc_body(j, acc):
        off = j * L
        return acc + sel_v[pl.ds(off, L)] * (one - rwd_v[pl.ds(off, L)])

    acc = lax.fori_loop(0, ROWS_PER_W // L, acc_body,
                        jnp.zeros((L,), jnp.float32), unroll=4)
    part_v[...] = acc
    pltpu.sync_copy(part_v, out_hbm.at[wid])


def kernel(prob, target, reward):
    tgt = target.astype(jnp.int32)
    rwd = reward.astype(jnp.float32)
    prob_phys = prob.reshape(128, 128, 125, 8).transpose(2, 0, 3, 1).reshape(-1)
    partials = _gan_loss_sc(prob_phys, tgt, rwd)
    return jnp.sum(partials) * (-1.0 / N)


# speedup vs baseline: 6.4077x; 1.0149x over previous
import functools

import jax
import jax.numpy as jnp
from jax import lax
from jax.experimental import pallas as pl
from jax.experimental.pallas import tpu as pltpu
from jax.experimental.pallas import tpu_sc as plsc

N = 16384
C = 1000
L = 16
NC = 2
NS = 16
NW = NC * NS
ROWS_PER_W = N // NW
CHUNK = 128
NCHUNK = ROWS_PER_W // CHUNK

_mesh = plsc.VectorSubcoreMesh(core_axis_name="c", subcore_axis_name="s")


@functools.partial(
    pl.kernel,
    mesh=_mesh,
    out_type=jax.ShapeDtypeStruct((NW, L), jnp.float32),
    scratch_types=[
        pltpu.VMEM((ROWS_PER_W,), jnp.int32),
        pltpu.VMEM((ROWS_PER_W,), jnp.float32),
        pltpu.VMEM((ROWS_PER_W,), jnp.int32),
        pltpu.VMEM((ROWS_PER_W,), jnp.float32),
        pltpu.VMEM((L,), jnp.float32),
        pltpu.SemaphoreType.DMA,
        pltpu.SemaphoreType.DMA,
        pltpu.SemaphoreType.DMA,
        pltpu.SemaphoreType.DMA,
        pltpu.SemaphoreType.DMA,
    ],
)
def _gan_loss_sc(prob_flat_hbm, tgt_hbm, rwd_hbm, out_hbm,
                 tgt_v, rwd_v, idx_v, sel_v, part_v, sem_in,
                 sg0, sg1, sg2, sg3):
    sem_g = (sg0, sg1, sg2, sg3)
    sid = lax.axis_index("s")
    cid = lax.axis_index("c")
    wid = sid * NC + cid
    base = wid * ROWS_PER_W

    half = ROWS_PER_W // 2
    cp_t0 = pltpu.async_copy(
        tgt_hbm.at[pl.ds(base, half)], tgt_v.at[pl.ds(0, half)], sem_in)
    cp_t1 = pltpu.async_copy(
        tgt_hbm.at[pl.ds(base + half, half)], tgt_v.at[pl.ds(half, half)], sem_in)
    cp_r = pltpu.async_copy(rwd_hbm.at[pl.ds(base, ROWS_PER_W)], rwd_v, sem_in)
    cp_t0.wait()

    lane = lax.broadcasted_iota(jnp.int32, (L,), 0)
    gathers = []
    for k in range(NCHUNK):
        if k == NCHUNK // 2:
            cp_t1.wait()

        def idx_body(j, _, k=k):
            off = k * CHUNK + j * L
            s = ((base + off) >> 7) * 1024 + ((base + off) & 127)
            t = tgt_v[pl.ds(off, L)]
            idx_v[pl.ds(off, L)] = (
                (t >> 3) * 131072 + (t & 7) * 128 + s + lane
            )
            return 0
        lax.fori_loop(0, CHUNK // L, idx_body, 0, unroll=2)
        gathers.append(
            pltpu.async_copy(
                prob_flat_hbm.at[idx_v.at[pl.ds(k * CHUNK, CHUNK)]],
                sel_v.at[pl.ds(k * CHUNK, CHUNK)],
                sem_g[k],
            )
        )

    cp_r.wait()
    one = jnp.full((L,), 1.0 + 1e-6, jnp.float32)
    acc = jnp.zeros((L,), jnp.float32)
    for k in range(NCHUNK):
        gathers[k].wait()

        def acc_body(j, acc, k=k):
            off = k * CHUNK + j * L
            return acc + sel_v[pl.ds(off, L)] * (one - rwd_v[pl.ds(off, L)])

        acc = lax.fori_loop(0, CHUNK // L, acc_body, acc, unroll=4)
    part_v[...] = acc
    pltpu.sync_copy(part_v, out_hbm.at[wid])


def kernel(prob, target, reward):
    tgt = target.astype(jnp.int32)
    rwd = reward.astype(jnp.float32)
    prob_phys = prob.reshape(128, 128, 125, 8).transpose(2, 0, 3, 1).reshape(-1)
    partials = _gan_loss_sc(prob_phys, tgt, rwd)
    return jnp.sum(partials) * (-1.0 / N)
